# Initial kernel scaffold; baseline (speedup 1.0000x reference)
import jax
import jax.numpy as jnp
from jax.experimental import pallas as pl


def kernel(token_ids, field_ids, chrom_ids, bin_ids, offset_norm, token_table, field_table, chrom_table, bin_table, W1, b1, W2, b2):
    raise NotImplementedError("write your pallas kernel here")


# speedup vs baseline: 4.1947x; 4.1947x over previous
import functools

import jax
import jax.numpy as jnp
from jax import lax
from jax.experimental import pallas as pl
from jax.experimental.pallas import tpu as pltpu
from jax.experimental.pallas import tpu_sc as plsc

EMBED = 64
NC, NS, LANES = 2, 16, 16
NW = NC * NS
WIN = 128
NMLP = 15488
_SQRT_HALF = 0.70710678118654752


def _erf(x):
    a1, a2, a3, a4, a5 = (0.254829592, -0.284496736, 1.421413741,
                          -1.453152027, 1.061405429)
    p = 0.3275911
    ax = jnp.abs(x)
    t = 1.0 / (1.0 + p * ax)
    poly = ((((a5 * t + a4) * t + a3) * t + a2) * t + a1) * t
    y = 1.0 - poly * jnp.exp(-ax * ax)
    return jnp.sign(x) * y


def _build_fc_table(field_table, chrom_table):
    nf, nc = field_table.shape[0], chrom_table.shape[0]

    def body(f_ref, c_ref, o_ref):
        o_ref[...] = f_ref[...][:, None, :] + c_ref[...][None, :, :]

    out = pl.pallas_call(
        body,
        out_shape=jax.ShapeDtypeStruct((nf, nc, EMBED), jnp.float32),
    )(field_table, chrom_table)
    return out.reshape(nf * nc, EMBED)


def _build_mlp_table(W1, b1, W2, b2):
    def body(w1_ref, b1_ref, w2_ref, b2_ref, o_ref):
        rows = lax.broadcasted_iota(jnp.int32, (NMLP, 1), 0)
        e = (rows >> 10) & 0x1F
        m = rows & 1023
        scale = lax.bitcast_convert_type(
            jnp.where(e == 0, jnp.int32(103 << 23), (e + 102) << 23),
            jnp.float32)
        mant = jnp.where(e == 0, m, m + 1024).astype(jnp.float32)
        x = mant * scale
        h = x * w1_ref[...] + b1_ref[...]
        h = h * 0.5 * (1.0 + _erf(h * _SQRT_HALF))
        o_ref[...] = (
            jnp.dot(h, w2_ref[...], preferred_element_type=jnp.float32)
            + b2_ref[...])

    return pl.pallas_call(
        body,
        out_shape=jax.ShapeDtypeStruct((NMLP, EMBED), jnp.float32),
    )(W1, b1.reshape(1, EMBED), W2, b2.reshape(1, EMBED))


def _sc_gather_sum(tok_idx, fc_idx, bin_idx, mlp_idx,
                   tok_tab, fc_tab, bin_tab, mlp_tab):
    n = tok_idx.shape[0]
    per_w = n // NW
    steps = per_w // WIN
    mesh = plsc.VectorSubcoreMesh(core_axis_name="c", subcore_axis_name="s")

    @functools.partial(
        pl.kernel,
        out_type=jax.ShapeDtypeStruct((n, EMBED), jnp.float32),
        mesh=mesh,
        compiler_params=pltpu.CompilerParams(use_tc_tiling_on_sc=False),
        scratch_types=[
            pltpu.VMEM((4, WIN), jnp.int32),
            pltpu.VMEM((WIN, EMBED), jnp.float32),
            pltpu.VMEM((WIN, EMBED), jnp.float32),
            pltpu.VMEM((WIN, EMBED), jnp.float32),
            pltpu.VMEM((WIN, EMBED), jnp.float32),
            pltpu.SemaphoreType.DMA,
            pltpu.SemaphoreType.DMA,
            pltpu.SemaphoreType.DMA,
            pltpu.SemaphoreType.DMA,
        ],
    )
    def k(ti_hbm, fi_hbm, bi_hbm, mi_hbm, tt_hbm, ft_hbm, bt_hbm, mt_hbm,
          out_hbm, idx_v, acc_v, fcr_v, binr_v, mlpr_v, s0, s1, s2, s3):
        wid = lax.axis_index("s") * NC + lax.axis_index("c")
        base = wid * per_w

        @pl.loop(0, steps)
        def _(step):
            off = base + step * WIN
            pltpu.sync_copy(ti_hbm.at[pl.ds(off, WIN)], idx_v.at[0])
            pltpu.sync_copy(fi_hbm.at[pl.ds(off, WIN)], idx_v.at[1])
            pltpu.sync_copy(bi_hbm.at[pl.ds(off, WIN)], idx_v.at[2])
            pltpu.sync_copy(mi_hbm.at[pl.ds(off, WIN)], idx_v.at[3])
            g0 = pltpu.async_copy(tt_hbm.at[idx_v.at[0]], acc_v, s0)
            g1 = pltpu.async_copy(ft_hbm.at[idx_v.at[1]], fcr_v, s1)
            g2 = pltpu.async_copy(bt_hbm.at[idx_v.at[2]], binr_v, s2)
            g3 = pltpu.async_copy(mt_hbm.at[idx_v.at[3]], mlpr_v, s3)
            g0.wait()
            g1.wait()
            g2.wait()
            g3.wait()

            @pl.loop(0, WIN)
            def _(r):
                for kk in range(EMBED // LANES):
                    sl = pl.ds(kk * LANES, LANES)
                    acc_v[r, sl] = ((acc_v[r, sl] + fcr_v[r, sl])
                                    + (binr_v[r, sl] + mlpr_v[r, sl]))

            pltpu.sync_copy(acc_v, out_hbm.at[pl.ds(off, WIN)])

    return k(tok_idx, fc_idx, bin_idx, mlp_idx, tok_tab, fc_tab, bin_tab,
             mlp_tab)


def kernel(token_ids, field_ids, chrom_ids, bin_ids, offset_norm,
           token_table, field_table, chrom_table, bin_table,
           W1, b1, W2, b2):
    shape = offset_norm.shape
    nc = chrom_table.shape[0]

    tok_idx = token_ids.reshape(-1).astype(jnp.int32)
    fc_idx = (field_ids.astype(jnp.int32) * nc
              + chrom_ids.astype(jnp.int32)).reshape(-1)
    bin_idx = bin_ids.reshape(-1).astype(jnp.int32)
    mlp_idx = lax.bitcast_convert_type(
        offset_norm.astype(jnp.float16), jnp.uint16).astype(jnp.int32)
    mlp_idx = jnp.clip(mlp_idx.reshape(-1), 0, NMLP - 1)

    fc_tab = _build_fc_table(field_table, chrom_table)
    mlp_tab = _build_mlp_table(W1, b1, W2, b2)

    out = _sc_gather_sum(tok_idx, fc_idx, bin_idx, mlp_idx,
                         token_table, fc_tab, bin_table, mlp_tab)
    return out.reshape(shape + (EMBED,))


# speedup vs baseline: 5.6185x; 1.3394x over previous
import functools

import jax
import jax.numpy as jnp
from jax import lax
from jax.experimental import pallas as pl
from jax.experimental.pallas import tpu as pltpu
from jax.experimental.pallas import tpu_sc as plsc

EMBED = 64
NC, NS, LANES = 2, 16, 16
NW = NC * NS
S0, S1 = 128, 72
WIN = S0 + S1
NMLP = 15488
_SQRT_HALF = 0.70710678118654752


def _erf(x):
    a1, a2, a3, a4, a5 = (0.254829592, -0.284496736, 1.421413741,
                          -1.453152027, 1.061405429)
    p = 0.3275911
    ax = jnp.abs(x)
    t = 1.0 / (1.0 + p * ax)
    poly = ((((a5 * t + a4) * t + a3) * t + a2) * t + a1) * t
    y = 1.0 - poly * jnp.exp(-ax * ax)
    return jnp.sign(x) * y


def _build_fc_table(field_table, chrom_table):
    nf, nc = field_table.shape[0], chrom_table.shape[0]

    def body(f_ref, c_ref, o_ref):
        o_ref[...] = f_ref[...][:, None, :] + c_ref[...][None, :, :]

    out = pl.pallas_call(
        body,
        out_shape=jax.ShapeDtypeStruct((nf, nc, EMBED), jnp.float32),
    )(field_table, chrom_table)
    return out.reshape(nf * nc, EMBED)


def _build_mlp_table(W1, b1, W2, b2):
    def body(w1_ref, b1_ref, w2_ref, b2_ref, o_ref):
        rows = lax.broadcasted_iota(jnp.int32, (NMLP, 1), 0)
        e = (rows >> 10) & 0x1F
        m = rows & 1023
        scale = lax.bitcast_convert_type(
            jnp.where(e == 0, jnp.int32(103 << 23), (e + 102) << 23),
            jnp.float32)
        mant = jnp.where(e == 0, m, m + 1024).astype(jnp.float32)
        x = mant * scale
        h = x * w1_ref[...] + b1_ref[...]
        h = h * 0.5 * (1.0 + _erf(h * _SQRT_HALF))
        o_ref[...] = (
            jnp.dot(h, w2_ref[...], preferred_element_type=jnp.float32)
            + b2_ref[...])

    return pl.pallas_call(
        body,
        out_shape=jax.ShapeDtypeStruct((NMLP, EMBED), jnp.float32),
    )(W1, b1.reshape(1, EMBED), W2, b2.reshape(1, EMBED))


def _sc_gather_sum(idx_pack, tok_tab, fc_tab, bin_tab, mlp_tab, b_dim, l_dim):
    nwin = idx_pack.shape[0]
    steps = nwin // NW
    mesh = plsc.VectorSubcoreMesh(core_axis_name="c", subcore_axis_name="s")

    @functools.partial(
        pl.kernel,
        out_type=jax.ShapeDtypeStruct((b_dim, l_dim, EMBED), jnp.float32),
        mesh=mesh,
        compiler_params=pltpu.CompilerParams(use_tc_tiling_on_sc=False),
        scratch_types=[
            pltpu.VMEM((8, 128), jnp.int32),
            pltpu.VMEM((8, 128), jnp.int32),
            pltpu.VMEM((WIN, EMBED), jnp.float32),
            pltpu.VMEM((WIN, EMBED), jnp.float32),
            pltpu.VMEM((WIN, EMBED), jnp.float32),
            pltpu.VMEM((WIN, EMBED), jnp.float32),
            pltpu.VMEM((WIN, EMBED), jnp.float32),
            pltpu.VMEM((WIN, EMBED), jnp.float32),
            pltpu.VMEM((WIN, EMBED), jnp.float32),
            pltpu.VMEM((WIN, EMBED), jnp.float32),
            pltpu.SemaphoreType.DMA,
            pltpu.SemaphoreType.DMA,
            pltpu.SemaphoreType.DMA,
            pltpu.SemaphoreType.DMA,
            pltpu.SemaphoreType.DMA,
            pltpu.SemaphoreType.DMA,
        ],
    )
    def k(ip_hbm, tt_hbm, ft_hbm, bt_hbm, mt_hbm, out_hbm,
          idx0, idx1, a0, f0, b0, m0, a1, f1, b1_, m1,
          gI0, gI1, gG0, gG1, gO0, gO1):
        wid = lax.axis_index("s") * NC + lax.axis_index("c")
        base = wid * steps
        idx_v = (idx0, idx1)
        bufs = ((a0, f0, b0, m0), (a1, f1, b1_, m1))
        semI = (gI0, gI1)
        semG = (gG0, gG1)
        semO = (gO0, gO1)
        tabs = (tt_hbm, ft_hbm, bt_hbm, mt_hbm)

        def idx_copy(w, p):
            return pltpu.make_async_copy(ip_hbm.at[base + w], idx_v[p],
                                         semI[p])

        def gathers(p):
            cps = []
            for t in range(4):
                for h, (off, sz) in enumerate(((0, S0), (S0, S1))):
                    cps.append(pltpu.make_async_copy(
                        tabs[t].at[idx_v[p].at[2 * t + h, pl.ds(0, sz)]],
                        bufs[p][t].at[pl.ds(off, sz)],
                        semG[p]))
            return cps

        def out_copy(w, p):
            return pltpu.make_async_copy(bufs[p][0], out_hbm.at[base + w],
                                         semO[p])

        idx_copy(0, 0).start()
        idx_copy(0, 0).wait()
        for cp in gathers(0):
            cp.start()
        idx_copy(1, 1).start()

        def half(w, cur, nxt):
            @pl.when(w + 1 < steps)
            def _():
                idx_copy(w + 1, nxt).wait()

                @pl.when(w >= 1)
                def _():
                    out_copy(w - 1, nxt).wait()
                for cp in gathers(nxt):
                    cp.start()

            for cp in gathers(cur):
                cp.wait()

            @pl.when(w + 2 < steps)
            def _():
                idx_copy(w + 2, cur).start()

            acc, fcr, binr, mlpr = bufs[cur]

            @pl.loop(0, WIN, step=8)
            def _(r0):
                for dr in range(8):
                    for kk in range(EMBED // LANES):
                        sl = pl.ds(kk * LANES, LANES)
                        acc[r0 + dr, sl] = (
                            (acc[r0 + dr, sl] + fcr[r0 + dr, sl])
                            + (binr[r0 + dr, sl] + mlpr[r0 + dr, sl]))

            out_copy(w, cur).start()

        @pl.loop(0, steps, step=2)
        def _(j):
            half(j, 0, 1)
            half(j + 1, 1, 0)

        out_copy(steps - 2, 0).wait()
        out_copy(steps - 1, 1).wait()

    return k(idx_pack, tok_tab, fc_tab, bin_tab, mlp_tab)


def kernel(token_ids, field_ids, chrom_ids, bin_ids, offset_norm,
           token_table, field_table, chrom_table, bin_table,
           W1, b1, W2, b2):
    b_dim, l_dim = offset_norm.shape
    nc = chrom_table.shape[0]

    tok_idx = token_ids.reshape(-1).astype(jnp.int32)
    fc_idx = (field_ids.astype(jnp.int32) * nc
              + chrom_ids.astype(jnp.int32)).reshape(-1)
    bin_idx = bin_ids.reshape(-1).astype(jnp.int32)
    mlp_idx = lax.bitcast_convert_type(
        offset_norm.astype(jnp.float16), jnp.uint16).astype(jnp.int32)
    mlp_idx = jnp.clip(mlp_idx.reshape(-1), 0, NMLP - 1)

    stacked = jnp.stack([tok_idx, fc_idx, bin_idx, mlp_idx])
    per = stacked.reshape(4, b_dim, WIN)
    first = per[:, :, :S0]
    second = jnp.pad(per[:, :, S0:], ((0, 0), (0, 0), (0, 128 - S1)))
    rows = jnp.stack([first, second], axis=2)
    idx_pack = rows.transpose(1, 0, 2, 3).reshape(b_dim, 8, 128)

    fc_tab = _build_fc_table(field_table, chrom_table)
    mlp_tab = _build_mlp_table(W1, b1, W2, b2)

    return _sc_gather_sum(idx_pack, token_table, fc_tab, bin_table, mlp_tab,
                          b_dim, l_dim)


# speedup vs baseline: 5.8738x; 1.0454x over previous
import functools

import jax
import jax.numpy as jnp
from jax import lax
from jax.experimental import pallas as pl
from jax.experimental.pallas import tpu as pltpu
from jax.experimental.pallas import tpu_sc as plsc

EMBED = 64
NC, NS, LANES = 2, 16, 16
NW = NC * NS
S0, S1 = 128, 72
WIN = S0 + S1
NMLP = 15488
_SQRT_HALF = 0.70710678118654752


def _erf(x):
    a1, a2, a3, a4, a5 = (0.254829592, -0.284496736, 1.421413741,
                          -1.453152027, 1.061405429)
    p = 0.3275911
    ax = jnp.abs(x)
    t = 1.0 / (1.0 + p * ax)
    poly = ((((a5 * t + a4) * t + a3) * t + a2) * t + a1) * t
    y = 1.0 - poly * jnp.exp(-ax * ax)
    return jnp.sign(x) * y


def _build_fc_table(field_table, chrom_table):
    nf, nc = field_table.shape[0], chrom_table.shape[0]

    def body(f_ref, c_ref, o_ref):
        o_ref[...] = f_ref[...][:, None, :] + c_ref[...][None, :, :]

    out = pl.pallas_call(
        body,
        out_shape=jax.ShapeDtypeStruct((nf, nc, EMBED), jnp.float32),
    )(field_table, chrom_table)
    return out.reshape(nf * nc, EMBED)


def _build_mlp_table(W1, b1, W2, b2):
    def body(w1_ref, b1_ref, w2_ref, b2_ref, o_ref):
        rows = lax.broadcasted_iota(jnp.int32, (NMLP, 1), 0)
        e = (rows >> 10) & 0x1F
        m = rows & 1023
        scale = lax.bitcast_convert_type(
            jnp.where(e == 0, jnp.int32(103 << 23), (e + 102) << 23),
            jnp.float32)
        mant = jnp.where(e == 0, m, m + 1024).astype(jnp.float32)
        x = mant * scale
        h = x * w1_ref[...] + b1_ref[...]
        h = h * 0.5 * (1.0 + _erf(h * _SQRT_HALF))
        o_ref[...] = (
            jnp.dot(h, w2_ref[...], preferred_element_type=jnp.float32)
            + b2_ref[...])

    return pl.pallas_call(
        body,
        out_shape=jax.ShapeDtypeStruct((NMLP, EMBED), jnp.float32),
    )(W1, b1.reshape(1, EMBED), W2, b2.reshape(1, EMBED))


def _sc_gather_sum(idx_pack, tok_tab, fc_tab, bin_tab, mlp_tab, b_dim, l_dim):
    nwin = idx_pack.shape[0]
    steps = nwin // NW
    mesh = plsc.VectorSubcoreMesh(core_axis_name="c", subcore_axis_name="s")

    @functools.partial(
        pl.kernel,
        out_type=jax.ShapeDtypeStruct((b_dim, l_dim, 2 * EMBED), jnp.float32),
        mesh=mesh,
        compiler_params=pltpu.CompilerParams(use_tc_tiling_on_sc=False),
        scratch_types=[
            pltpu.VMEM((8, 128), jnp.int32),
            pltpu.VMEM((8, 128), jnp.int32),
            pltpu.VMEM((WIN, 2 * EMBED), jnp.float32),
            pltpu.VMEM((WIN, EMBED), jnp.float32),
            pltpu.VMEM((WIN, EMBED), jnp.float32),
            pltpu.VMEM((WIN, EMBED), jnp.float32),
            pltpu.VMEM((WIN, 2 * EMBED), jnp.float32),
            pltpu.VMEM((WIN, EMBED), jnp.float32),
            pltpu.VMEM((WIN, EMBED), jnp.float32),
            pltpu.VMEM((WIN, EMBED), jnp.float32),
            pltpu.SemaphoreType.DMA,
            pltpu.SemaphoreType.DMA,
            pltpu.SemaphoreType.DMA,
            pltpu.SemaphoreType.DMA,
            pltpu.SemaphoreType.DMA,
            pltpu.SemaphoreType.DMA,
        ],
    )
    def k(ip_hbm, tt_hbm, ft_hbm, bt_hbm, mt_hbm, out_hbm,
          idx0, idx1, a0, f0, b0, m0, a1, f1, b1_, m1,
          gI0, gI1, gG0, gG1, gO0, gO1):
        wid = lax.axis_index("s") * NC + lax.axis_index("c")
        base = wid * steps
        idx_v = (idx0, idx1)
        bufs = ((a0, f0, b0, m0), (a1, f1, b1_, m1))
        semI = (gI0, gI1)
        semG = (gG0, gG1)
        semO = (gO0, gO1)
        tabs = (tt_hbm, ft_hbm, bt_hbm, mt_hbm)

        def idx_copy(w, p):
            return pltpu.make_async_copy(ip_hbm.at[base + w], idx_v[p],
                                         semI[p])

        def gathers(p):
            cps = []
            for t in range(4):
                for h, (off, sz) in enumerate(((0, S0), (S0, S1))):
                    cps.append(pltpu.make_async_copy(
                        tabs[t].at[idx_v[p].at[2 * t + h, pl.ds(0, sz)]],
                        bufs[p][t].at[pl.ds(off, sz)],
                        semG[p]))
            return cps

        def out_copy(w, p):
            return pltpu.make_async_copy(bufs[p][0], out_hbm.at[base + w],
                                         semO[p])

        idx_copy(0, 0).start()
        idx_copy(0, 0).wait()
        for cp in gathers(0):
            cp.start()
        idx_copy(1, 1).start()

        def half(w, cur, nxt):
            @pl.when(w + 1 < steps)
            def _():
                idx_copy(w + 1, nxt).wait()

                @pl.when(w >= 1)
                def _():
                    out_copy(w - 1, nxt).wait()
                for cp in gathers(nxt):
                    cp.start()

            for cp in gathers(cur):
                cp.wait()

            @pl.when(w + 2 < steps)
            def _():
                idx_copy(w + 2, cur).start()

            acc, fcr, binr, mlpr = bufs[cur]

            @pl.loop(0, WIN, step=8)
            def _(r0):
                for dr in range(8):
                    for kk in range(EMBED // LANES):
                        sl = pl.ds(kk * LANES, LANES)
                        acc[r0 + dr, sl] = (
                            (acc[r0 + dr, sl] + fcr[r0 + dr, sl])
                            + (binr[r0 + dr, sl] + mlpr[r0 + dr, sl]))

            out_copy(w, cur).start()

        @pl.loop(0, steps, step=2)
        def _(j):
            half(j, 0, 1)
            half(j + 1, 1, 0)

        out_copy(steps - 2, 0).wait()
        out_copy(steps - 1, 1).wait()

    return k(idx_pack, tok_tab, fc_tab, bin_tab, mlp_tab)


def kernel(token_ids, field_ids, chrom_ids, bin_ids, offset_norm,
           token_table, field_table, chrom_table, bin_table,
           W1, b1, W2, b2):
    b_dim, l_dim = offset_norm.shape
    nc = chrom_table.shape[0]

    tok_idx = token_ids.reshape(-1).astype(jnp.int32)
    fc_idx = (field_ids.astype(jnp.int32) * nc
              + chrom_ids.astype(jnp.int32)).reshape(-1)
    bin_idx = bin_ids.reshape(-1).astype(jnp.int32)
    mlp_idx = lax.bitcast_convert_type(
        offset_norm.astype(jnp.float16), jnp.uint16).astype(jnp.int32)
    mlp_idx = jnp.clip(mlp_idx.reshape(-1), 0, NMLP - 1)

    stacked = jnp.stack([tok_idx, fc_idx, bin_idx, mlp_idx])
    per = stacked.reshape(4, b_dim, WIN)
    first = per[:, :, :S0]
    second = jnp.pad(per[:, :, S0:], ((0, 0), (0, 0), (0, 128 - S1)))
    rows = jnp.stack([first, second], axis=2)
    idx_pack = rows.transpose(1, 0, 2, 3).reshape(b_dim, 8, 128)

    fc_tab = _build_fc_table(field_table, chrom_table)
    mlp_tab = _build_mlp_table(W1, b1, W2, b2)

    tok128 = jnp.pad(token_table, ((0, 0), (0, 2 * EMBED - token_table.shape[1])))

    out = _sc_gather_sum(idx_pack, tok128, fc_tab, bin_table, mlp_tab,
                         b_dim, l_dim)
    return out[:, :, :EMBED]
